# Initial kernel scaffold; baseline (speedup 1.0000x reference)
import jax
import jax.numpy as jnp
from jax.experimental import pallas as pl


def kernel(c, o, neg, V_emb, U_emb):
    raise NotImplementedError("write your pallas kernel here")


# speedup vs baseline: 4.1342x; 4.1342x over previous
import jax
import jax.numpy as jnp
from jax import lax
from jax.experimental import pallas as pl
from jax.experimental.pallas import tpu as pltpu
from jax.experimental.pallas import tpu_sc as plsc

_NC = 2
_NS = 16
_L = 16
_LN2 = 0.6931471805599453
_SQRT2 = 1.4142135381698608


def _logsig_eps(s):
    z = jnp.exp(-s)
    y = 1.0 / (1.0 + z) + 1e-9
    bits = lax.bitcast_convert_type(y, jnp.int32)
    e = (bits >> 23) - 127
    m = lax.bitcast_convert_type((bits & 0x007FFFFF) | 0x3F800000, jnp.float32)
    big = m > _SQRT2
    m = jnp.where(big, m * 0.5, m)
    e = jnp.where(big, e + 1, e)
    t = (m - 1.0) / (m + 1.0)
    t2 = t * t
    p = 1.0 + t2 * (0.3333333333 + t2 * (0.2 + t2 * (0.142857143 + t2 * 0.111111111)))
    return e.astype(jnp.float32) * _LN2 + 2.0 * t * p


def _make_sgns_kernel(B, K, V, D):
    NW = _NC * _NS
    assert B % NW == 0
    BPW = B // NW
    CH = 16 if BPW % 16 == 0 else BPW
    assert BPW % CH == 0 and D % _L == 0
    NCHUNK = BPW // CH
    KP1 = K + 1
    NDOT = CH * KP1
    assert NDOT % _L == 0
    ND = D // _L
    assert (CH * K) % _L == 0
    mesh = plsc.VectorSubcoreMesh(
        core_axis_name="cores", subcore_axis_name="subcores",
        num_cores=_NC, num_subcores=_NS)

    def body(c_hbm, o_hbm, neg_hbm, v_hbm, u_hbm, out_hbm,
             c_idx, o_idx, n_idx, vc_b, uo_b, un_b, dots, acc, sem):
        wid = lax.axis_index("subcores") * _NC + lax.axis_index("cores")
        b0 = wid * BPW
        pltpu.sync_copy(c_hbm.at[pl.ds(b0, BPW)], c_idx)
        pltpu.sync_copy(o_hbm.at[pl.ds(b0, BPW)], o_idx)
        pltpu.sync_copy(neg_hbm.at[pl.ds(b0 * K, BPW * K)], n_idx)
        acc[...] = jnp.zeros((_L,), jnp.float32)

        def chunk(i, _):
            cb = i * CH
            nb = i * (CH * K)
            waits = [
                pltpu.async_copy(v_hbm.at[c_idx[pl.ds(cb, CH)]], vc_b, sem),
                pltpu.async_copy(u_hbm.at[o_idx[pl.ds(cb, CH)]], uo_b, sem),
            ]
            for g in range(CH * K // _L):
                waits.append(pltpu.async_copy(
                    u_hbm.at[n_idx[pl.ds(nb + g * _L, _L)]],
                    un_b.at[pl.ds(g * _L, _L)], sem))
            for w in waits:
                w.wait()

            lane = lax.iota(jnp.int32, _L)
            m15 = lane == (_L - 1)

            def put(idx, vec):
                plsc.store_scatter(
                    dots, [jnp.full((_L,), idx, jnp.int32)], vec, mask=m15)

            def per_b(b, _):
                vc = [vc_b[b, pl.ds(_L * j, _L)] for j in range(ND)]
                pp = vc[0] * uo_b[b, pl.ds(0, _L)]
                for j in range(1, ND):
                    pp = pp + vc[j] * uo_b[b, pl.ds(_L * j, _L)]
                put(b * KP1, plsc.cumsum(pp))
                for k in range(K):
                    r = b * K + k
                    nn = vc[0] * un_b[r, pl.ds(0, _L)]
                    for j in range(1, ND):
                        nn = nn + vc[j] * un_b[r, pl.ds(_L * j, _L)]
                    put(b * KP1 + 1 + k, -plsc.cumsum(nn))
                return 0

            lax.fori_loop(0, CH, per_b, 0)
            a = acc[...]
            for v in range(NDOT // _L):
                a = a + _logsig_eps(dots[pl.ds(_L * v, _L)])
            acc[...] = a
            return 0

        lax.fori_loop(0, NCHUNK, chunk, 0)
        pltpu.sync_copy(acc, out_hbm.at[wid])

    return pl.kernel(
        body,
        out_type=jax.ShapeDtypeStruct((NW, _L), jnp.float32),
        mesh=mesh,
        compiler_params=pltpu.CompilerParams(needs_layout_passes=False),
        scratch_types=[
            pltpu.VMEM((BPW,), jnp.int32),
            pltpu.VMEM((BPW,), jnp.int32),
            pltpu.VMEM((BPW * K,), jnp.int32),
            pltpu.VMEM((CH, D), jnp.float32),
            pltpu.VMEM((CH, D), jnp.float32),
            pltpu.VMEM((CH * K, D), jnp.float32),
            pltpu.VMEM((NDOT,), jnp.float32),
            pltpu.VMEM((_L,), jnp.float32),
            pltpu.SemaphoreType.DMA,
        ],
    )


def kernel(c, o, neg, V_emb, U_emb):
    (B,) = c.shape
    K = neg.shape[1]
    V, D = V_emb.shape
    c = c.astype(jnp.int32)
    o = o.astype(jnp.int32)
    negf = neg.reshape(-1).astype(jnp.int32)
    fn = _make_sgns_kernel(B, K, V, D)
    partials = fn(c, o, negf, V_emb, U_emb)
    return -(jnp.sum(partials) / B)


# speedup vs baseline: 5.0591x; 1.2237x over previous
import jax
import jax.numpy as jnp
from jax import lax
from jax.experimental import pallas as pl
from jax.experimental.pallas import tpu as pltpu
from jax.experimental.pallas import tpu_sc as plsc

_NC = 2
_NS = 16
_L = 16
_LN2 = 0.6931471805599453
_SQRT2 = 1.4142135381698608


def _logsig_eps(s):
    z = jnp.exp(-s)
    y = 1.0 / (1.0 + z) + 1e-9
    bits = lax.bitcast_convert_type(y, jnp.int32)
    e = (bits >> 23) - 127
    m = lax.bitcast_convert_type((bits & 0x007FFFFF) | 0x3F800000, jnp.float32)
    big = m > _SQRT2
    m = jnp.where(big, m * 0.5, m)
    e = jnp.where(big, e + 1, e)
    t = (m - 1.0) / (m + 1.0)
    t2 = t * t
    p = 1.0 + t2 * (0.3333333333 + t2 * (0.2 + t2 * (0.142857143 + t2 * 0.111111111)))
    return e.astype(jnp.float32) * _LN2 + 2.0 * t * p


def _make_sgns_kernel(B, K, V, D):
    NW = _NC * _NS
    assert B % NW == 0
    BPW = B // NW
    CH = 16 if BPW % 16 == 0 else BPW
    assert BPW % CH == 0 and D % _L == 0
    NCHUNK = BPW // CH
    KP1 = K + 1
    NDOT = CH * KP1
    assert NDOT % _L == 0
    ND = D // _L
    assert (CH * K) % _L == 0
    mesh = plsc.VectorSubcoreMesh(
        core_axis_name="cores", subcore_axis_name="subcores",
        num_cores=_NC, num_subcores=_NS)

    assert NCHUNK % 2 == 0
    NSTEP = NCHUNK // 2

    def body(c_hbm, o_hbm, neg_hbm, v_hbm, u_hbm, out_hbm,
             c_idx, o_idx, n_idx, vc_b0, uo_b0, un_b0, vc_b1, uo_b1, un_b1,
             dots, acc, sem0, sem1):
        wid = lax.axis_index("subcores") * _NC + lax.axis_index("cores")
        b0 = wid * BPW
        pltpu.sync_copy(c_hbm.at[pl.ds(b0, BPW)], c_idx)
        pltpu.sync_copy(o_hbm.at[pl.ds(b0, BPW)], o_idx)
        pltpu.sync_copy(neg_hbm.at[pl.ds(b0 * K, BPW * K)], n_idx)
        acc[...] = jnp.zeros((_L,), jnp.float32)

        def issue(i, vcb, uob, unb, sem):
            cb = i * CH
            nb = i * (CH * K)
            pltpu.async_copy(v_hbm.at[c_idx[pl.ds(cb, CH)]], vcb, sem)
            pltpu.async_copy(u_hbm.at[o_idx[pl.ds(cb, CH)]], uob, sem)
            for g in range(CH * K // _L):
                pltpu.async_copy(
                    u_hbm.at[n_idx[pl.ds(nb + g * _L, _L)]],
                    unb.at[pl.ds(g * _L, _L)], sem)

        def drain(vcb, uob, unb, sem):
            pltpu.make_async_copy(v_hbm.at[pl.ds(0, CH)], vcb, sem).wait()
            pltpu.make_async_copy(u_hbm.at[pl.ds(0, CH)], uob, sem).wait()
            pltpu.make_async_copy(u_hbm.at[pl.ds(0, CH * K)], unb, sem).wait()

        lane = lax.iota(jnp.int32, _L)
        m15 = lane == (_L - 1)

        def put(idx, vec):
            plsc.store_scatter(
                dots, [jnp.full((_L,), idx, jnp.int32)], vec, mask=m15)

        def compute(vcb, uob, unb):
            def per_b(b, _):
                vc = [vcb[b, pl.ds(_L * j, _L)] for j in range(ND)]
                pp = vc[0] * uob[b, pl.ds(0, _L)]
                for j in range(1, ND):
                    pp = pp + vc[j] * uob[b, pl.ds(_L * j, _L)]
                put(b * KP1, plsc.cumsum(pp))
                for k in range(K):
                    r = b * K + k
                    nn = vc[0] * unb[r, pl.ds(0, _L)]
                    for j in range(1, ND):
                        nn = nn + vc[j] * unb[r, pl.ds(_L * j, _L)]
                    put(b * KP1 + 1 + k, -plsc.cumsum(nn))
                return 0

            lax.fori_loop(0, CH, per_b, 0)
            a = acc[...]
            for v in range(NDOT // _L):
                a = a + _logsig_eps(dots[pl.ds(_L * v, _L)])
            acc[...] = a

        issue(0, vc_b0, uo_b0, un_b0, sem0)

        def step(s, _):
            issue(2 * s + 1, vc_b1, uo_b1, un_b1, sem1)
            drain(vc_b0, uo_b0, un_b0, sem0)
            compute(vc_b0, uo_b0, un_b0)

            @pl.when(s + 1 < NSTEP)
            def _():
                issue(2 * s + 2, vc_b0, uo_b0, un_b0, sem0)

            drain(vc_b1, uo_b1, un_b1, sem1)
            compute(vc_b1, uo_b1, un_b1)
            return 0

        lax.fori_loop(0, NSTEP, step, 0)
        pltpu.sync_copy(acc, out_hbm.at[wid])

    return pl.kernel(
        body,
        out_type=jax.ShapeDtypeStruct((NW, _L), jnp.float32),
        mesh=mesh,
        compiler_params=pltpu.CompilerParams(needs_layout_passes=False),
        scratch_types=[
            pltpu.VMEM((BPW,), jnp.int32),
            pltpu.VMEM((BPW,), jnp.int32),
            pltpu.VMEM((BPW * K,), jnp.int32),
            pltpu.VMEM((CH, D), jnp.float32),
            pltpu.VMEM((CH, D), jnp.float32),
            pltpu.VMEM((CH * K, D), jnp.float32),
            pltpu.VMEM((CH, D), jnp.float32),
            pltpu.VMEM((CH, D), jnp.float32),
            pltpu.VMEM((CH * K, D), jnp.float32),
            pltpu.VMEM((NDOT,), jnp.float32),
            pltpu.VMEM((_L,), jnp.float32),
            pltpu.SemaphoreType.DMA,
            pltpu.SemaphoreType.DMA,
        ],
    )


def kernel(c, o, neg, V_emb, U_emb):
    (B,) = c.shape
    K = neg.shape[1]
    V, D = V_emb.shape
    c = c.astype(jnp.int32)
    o = o.astype(jnp.int32)
    negf = neg.reshape(-1).astype(jnp.int32)
    fn = _make_sgns_kernel(B, K, V, D)
    partials = fn(c, o, negf, V_emb, U_emb)
    return -(jnp.sum(partials) / B)
